# Initial kernel scaffold; baseline (speedup 1.0000x reference)
import jax
import jax.numpy as jnp
from jax.experimental import pallas as pl


def kernel(x, edge_index, edge_attr, prob, weight, w_ih, w_hh, b_ih, b_hh, lin_w, lin_b):
    raise NotImplementedError("write your pallas kernel here")


# speedup vs baseline: 1.4409x; 1.4409x over previous
import functools

import jax
import jax.numpy as jnp
from jax import lax
from jax.experimental import pallas as pl
from jax.experimental.pallas import tpu as pltpu
from jax.experimental.pallas import tpu_sc as plsc

H = 1000
HP = 1024
D = 128
NFB = HP // D
N = 10000
NP = 10240
E = 320000
ECHUNK = 128
NTILES = 16
EPT = 157 * ECHUNK
EP = EPT * NTILES
STRIPE = NP // NTILES
BN = 256
BN0 = 1024


def _mm0_body(xb, wb, ob):
    ob[...] = jnp.dot(xb[...], wb[...], preferred_element_type=jnp.float32)[None]


def _mm0(xp, w0r):
    return pl.pallas_call(
        _mm0_body,
        grid=(NP // BN0, NFB),
        in_specs=[
            pl.BlockSpec((BN0, D), lambda i, j: (i, 0)),
            pl.BlockSpec((D, D), lambda i, j: (0, j)),
        ],
        out_specs=pl.BlockSpec((1, BN0, D), lambda i, j: (j, i, 0)),
        out_shape=jax.ShapeDtypeStruct((NFB, NP, D), jnp.float32),
    )(xp, w0r)


def _gru_math(gi, gh, hv):
    r = jax.nn.sigmoid(gi[:, :HP] + gh[:, :HP])
    z = jax.nn.sigmoid(gi[:, HP:2 * HP] + gh[:, HP:2 * HP])
    n = jnp.tanh(gi[:, 2 * HP:] + r * gh[:, 2 * HP:])
    return (1.0 - z) * n + z * hv


def _gates(aggb, hb, wih, whh, bi, bh):
    hv = hb[...]
    gh = jnp.dot(hv, whh[...], preferred_element_type=jnp.float32) + bh[...][0:1, :]
    wihv = wih[...]
    gi = bi[...][0:1, :]
    for k in range(NFB):
        gi = gi + jnp.dot(aggb[k], wihv[k * D:(k + 1) * D, :],
                          preferred_element_type=jnp.float32)
    return _gru_math(gi, gh, hv), hv


def _gru_next_body(aggb, hb, wih, whh, bi, bh, wn, hob, mob):
    hn, _ = _gates(aggb, hb, wih, whh, bi, bh)
    hob[...] = hn
    wnv = wn[...]
    for k in range(NFB):
        mob[k] = jnp.dot(hn, wnv[:, k * D:(k + 1) * D],
                         preferred_element_type=jnp.float32)


def _gru_next(agg3, h, wih, whh, bi, bh, wn):
    return pl.pallas_call(
        _gru_next_body,
        grid=(NP // BN,),
        in_specs=[
            pl.BlockSpec((NFB, BN, D), lambda i: (0, i, 0)),
            pl.BlockSpec((BN, HP), lambda i: (i, 0)),
            pl.BlockSpec((HP, 3 * HP), lambda i: (0, 0)),
            pl.BlockSpec((HP, 3 * HP), lambda i: (0, 0)),
            pl.BlockSpec((8, 3 * HP), lambda i: (0, 0)),
            pl.BlockSpec((8, 3 * HP), lambda i: (0, 0)),
            pl.BlockSpec((HP, HP), lambda i: (0, 0)),
        ],
        out_specs=[
            pl.BlockSpec((BN, HP), lambda i: (i, 0)),
            pl.BlockSpec((NFB, BN, D), lambda i: (0, i, 0)),
        ],
        out_shape=[
            jax.ShapeDtypeStruct((NP, HP), jnp.float32),
            jax.ShapeDtypeStruct((NFB, NP, D), jnp.float32),
        ],
    )(agg3, h, wih, whh, bi, bh, wn)


def _gru_final_body(aggb, hb, wih, whh, bi, bh, lp, ob):
    hn, _ = _gates(aggb, hb, wih, whh, bi, bh)
    ob[...] = jnp.dot(jnp.maximum(hn, 0.0), lp[...],
                      preferred_element_type=jnp.float32)


def _gru_final(agg3, h, wih, whh, bi, bh, lp):
    return pl.pallas_call(
        _gru_final_body,
        grid=(NP // BN,),
        in_specs=[
            pl.BlockSpec((NFB, BN, D), lambda i: (0, i, 0)),
            pl.BlockSpec((BN, HP), lambda i: (i, 0)),
            pl.BlockSpec((HP, 3 * HP), lambda i: (0, 0)),
            pl.BlockSpec((HP, 3 * HP), lambda i: (0, 0)),
            pl.BlockSpec((8, 3 * HP), lambda i: (0, 0)),
            pl.BlockSpec((8, 3 * HP), lambda i: (0, 0)),
            pl.BlockSpec((HP, D), lambda i: (0, 0)),
        ],
        out_specs=pl.BlockSpec((BN, D), lambda i: (i, 0)),
        out_shape=jax.ShapeDtypeStruct((NP, D), jnp.float32),
    )(agg3, h, wih, whh, bi, bh, lp)


@functools.cache
def _sc_agg_fn():
    mesh = plsc.VectorSubcoreMesh(core_axis_name="c", subcore_axis_name="s")

    @functools.partial(
        pl.kernel,
        mesh=mesh,
        out_type=jax.ShapeDtypeStruct((NFB * NP, D), jnp.float32),
        scratch_types=[
            pltpu.VMEM_SHARED((NP, D), jnp.float32),
            pltpu.VMEM((ECHUNK,), jnp.int32),
            pltpu.VMEM((ECHUNK,), jnp.int32),
            pltpu.VMEM((ECHUNK,), jnp.float32),
            pltpu.VMEM((ECHUNK, D), jnp.float32),
            pltpu.SemaphoreType.DMA,
        ],
    )
    def _sc_agg(m_hbm, src_hbm, dst_hbm, attr_hbm, zeros_hbm, agg_hbm,
                slab, idx_s, idx_d, attr_v, rows, sem):
        cid = lax.axis_index("c")
        sid = lax.axis_index("s")
        ebase0 = sid * EPT
        for j in range(NFB // 2):
            fb = cid * (NFB // 2) + j
            fboff = fb * NP
            pltpu.sync_copy(zeros_hbm, slab.at[pl.ds(sid * STRIPE, STRIPE)])
            plsc.subcore_barrier()

            def chunk_body(t, carry):
                base = ebase0 + t * ECHUNK
                pltpu.sync_copy(src_hbm.at[pl.ds(base, ECHUNK)], idx_s)
                pltpu.sync_copy(dst_hbm.at[pl.ds(base, ECHUNK)], idx_d)
                pltpu.sync_copy(attr_hbm.at[pl.ds(base, ECHUNK)], attr_v)
                for c in range(ECHUNK // 16):
                    sl = pl.ds(c * 16, 16)
                    idx_s[sl] = idx_s[sl] + fboff
                pltpu.async_copy(m_hbm.at[idx_s], rows, sem).wait()

                def mul_body(g, c2):
                    av = attr_v[pl.ds(g * 16, 16)]
                    for l in range(16):
                        e = g * 16 + l
                        sp = lax.gather(
                            av, jnp.full((16, 1), l, jnp.int32),
                            lax.GatherDimensionNumbers(
                                offset_dims=(), collapsed_slice_dims=(0,),
                                start_index_map=(0,)),
                            (1,), mode=lax.GatherScatterMode.PROMISE_IN_BOUNDS)
                        for c in range(D // 16):
                            sl = pl.ds(c * 16, 16)
                            rows[e, sl] = rows[e, sl] * sp
                    return c2

                lax.fori_loop(0, ECHUNK // 16, mul_body, 0)
                pltpu.sync_copy(rows, slab.at[idx_d], add=True)
                return carry

            lax.fori_loop(0, EPT // ECHUNK, chunk_body, 0)
            plsc.subcore_barrier()
            pltpu.sync_copy(slab.at[pl.ds(sid * STRIPE, STRIPE)],
                            agg_hbm.at[pl.ds(fboff + sid * STRIPE, STRIPE)])
            plsc.subcore_barrier()

    return _sc_agg


def _sc_agg_call(m2, srcp, dstp, attrp, zerosb):
    return _sc_agg_fn()(m2, srcp, dstp, attrp, zerosb)


def _gate_pack(w):
    wt = w.T.reshape(H, 3, H)
    wt = jnp.pad(wt, ((0, HP - H), (0, 0), (0, HP - H)))
    return wt.reshape(HP, 3 * HP)


def _bias_pack(b):
    bp = jnp.pad(b.reshape(3, H), ((0, 0), (0, HP - H))).reshape(3 * HP)
    return jnp.broadcast_to(bp, (8, 3 * HP))


def kernel(x, edge_index, edge_attr, prob, weight, w_ih, w_hh, b_ih, b_hh,
           lin_w, lin_b):
    f32 = jnp.float32
    xp = jnp.pad(x.astype(f32), ((0, NP - N), (0, 0)))
    h = jnp.pad(xp, ((0, 0), (0, HP - D)))
    w0r = jnp.pad(weight[0][:D, :], ((0, 0), (0, HP - H)))
    wnext = {
        1: jnp.pad(weight[1], ((0, HP - H), (0, HP - H))),
        2: jnp.pad(weight[2], ((0, HP - H), (0, HP - H))),
    }
    wih_t = _gate_pack(w_ih)
    whh_t = _gate_pack(w_hh)
    bi = _bias_pack(b_ih)
    bh = _bias_pack(b_hh)
    srcp = jnp.pad(edge_index[0].astype(jnp.int32), (0, EP - E))
    dstp = jnp.pad(edge_index[1].astype(jnp.int32), (0, EP - E))
    attrp = jnp.pad(edge_attr.astype(f32), (0, EP - E))
    zerosb = jnp.zeros((STRIPE, D), f32)
    linp = jnp.zeros((HP, D), f32).at[:H, 0].set(lin_w[0])

    m = _mm0(xp, w0r)
    for layer in (1, 2):
        agg = _sc_agg_call(m.reshape(NFB * NP, D), srcp, dstp, attrp, zerosb)
        h, m = _gru_next(agg.reshape(NFB, NP, D), h, wih_t, whh_t, bi, bh,
                         wnext[layer])
    agg = _sc_agg_call(m.reshape(NFB * NP, D), srcp, dstp, attrp, zerosb)
    out = _gru_final(agg.reshape(NFB, NP, D), h, wih_t, whh_t, bi, bh, linp)
    return out[:N, :1] + lin_b[0]


# speedup vs baseline: 1.7082x; 1.1855x over previous
import functools

import jax
import jax.numpy as jnp
from jax import lax
from jax.experimental import pallas as pl
from jax.experimental.pallas import tpu as pltpu
from jax.experimental.pallas import tpu_sc as plsc

H = 1000
HP = 1024
D = 128
NFB = HP // D
N = 10000
NP = 10240
E = 320000
ECHUNK = 48
NTILES = 16
NCHUNK = 420
EPT = NCHUNK * ECHUNK
EP = EPT * NTILES
SLAB_N = 10112
STRIPE = SLAB_N // NTILES
BN = 256
BN0 = 1024


def _mm0_body(xb, wb, ob):
    ob[...] = jnp.dot(xb[...], wb[...], preferred_element_type=jnp.float32)[None]


def _mm0(xp, w0r):
    return pl.pallas_call(
        _mm0_body,
        grid=(NP // BN0, NFB),
        in_specs=[
            pl.BlockSpec((BN0, D), lambda i, j: (i, 0)),
            pl.BlockSpec((D, D), lambda i, j: (0, j)),
        ],
        out_specs=pl.BlockSpec((1, BN0, D), lambda i, j: (j, i, 0)),
        out_shape=jax.ShapeDtypeStruct((NFB, NP, D), jnp.float32),
    )(xp, w0r)


def _gru_math(gi, gh, hv):
    r = jax.nn.sigmoid(gi[:, :HP] + gh[:, :HP])
    z = jax.nn.sigmoid(gi[:, HP:2 * HP] + gh[:, HP:2 * HP])
    n = jnp.tanh(gi[:, 2 * HP:] + r * gh[:, 2 * HP:])
    return (1.0 - z) * n + z * hv


def _gates(aggb, hb, wih, whh, bi, bh):
    hv = hb[...]
    gh = jnp.dot(hv, whh[...], preferred_element_type=jnp.float32) + bh[...][0:1, :]
    wihv = wih[...]
    gi = bi[...][0:1, :]
    for k in range(NFB):
        gi = gi + jnp.dot(aggb[k], wihv[k * D:(k + 1) * D, :],
                          preferred_element_type=jnp.float32)
    return _gru_math(gi, gh, hv), hv


def _gru_next_body(aggb, hb, wih, whh, bi, bh, wn, hob, mob):
    hn, _ = _gates(aggb, hb, wih, whh, bi, bh)
    hob[...] = hn
    wnv = wn[...]
    for k in range(NFB):
        mob[k] = jnp.dot(hn, wnv[:, k * D:(k + 1) * D],
                         preferred_element_type=jnp.float32)


def _gru_next(agg3, h, wih, whh, bi, bh, wn):
    return pl.pallas_call(
        _gru_next_body,
        grid=(NP // BN,),
        in_specs=[
            pl.BlockSpec((NFB, BN, D), lambda i: (0, i, 0)),
            pl.BlockSpec((BN, HP), lambda i: (i, 0)),
            pl.BlockSpec((HP, 3 * HP), lambda i: (0, 0)),
            pl.BlockSpec((HP, 3 * HP), lambda i: (0, 0)),
            pl.BlockSpec((8, 3 * HP), lambda i: (0, 0)),
            pl.BlockSpec((8, 3 * HP), lambda i: (0, 0)),
            pl.BlockSpec((HP, HP), lambda i: (0, 0)),
        ],
        out_specs=[
            pl.BlockSpec((BN, HP), lambda i: (i, 0)),
            pl.BlockSpec((NFB, BN, D), lambda i: (0, i, 0)),
        ],
        out_shape=[
            jax.ShapeDtypeStruct((NP, HP), jnp.float32),
            jax.ShapeDtypeStruct((NFB, NP, D), jnp.float32),
        ],
    )(agg3, h, wih, whh, bi, bh, wn)


def _gru_final_body(aggb, hb, wih, whh, bi, bh, lp, ob):
    hn, _ = _gates(aggb, hb, wih, whh, bi, bh)
    ob[...] = jnp.dot(jnp.maximum(hn, 0.0), lp[...],
                      preferred_element_type=jnp.float32)


def _gru_final(agg3, h, wih, whh, bi, bh, lp):
    return pl.pallas_call(
        _gru_final_body,
        grid=(NP // BN,),
        in_specs=[
            pl.BlockSpec((NFB, BN, D), lambda i: (0, i, 0)),
            pl.BlockSpec((BN, HP), lambda i: (i, 0)),
            pl.BlockSpec((HP, 3 * HP), lambda i: (0, 0)),
            pl.BlockSpec((HP, 3 * HP), lambda i: (0, 0)),
            pl.BlockSpec((8, 3 * HP), lambda i: (0, 0)),
            pl.BlockSpec((8, 3 * HP), lambda i: (0, 0)),
            pl.BlockSpec((HP, D), lambda i: (0, 0)),
        ],
        out_specs=pl.BlockSpec((BN, D), lambda i: (i, 0)),
        out_shape=jax.ShapeDtypeStruct((NP, D), jnp.float32),
    )(agg3, h, wih, whh, bi, bh, lp)


@functools.cache
def _sc_agg_fn():
    mesh = plsc.VectorSubcoreMesh(core_axis_name="c", subcore_axis_name="s")

    @functools.partial(
        pl.kernel,
        mesh=mesh,
        out_type=jax.ShapeDtypeStruct((NFB * NP, D), jnp.float32),
        scratch_types=[
            pltpu.VMEM_SHARED((SLAB_N, D), jnp.float32),
        ] + [pltpu.VMEM((ECHUNK,), jnp.int32) for _ in range(4)]
          + [pltpu.VMEM((ECHUNK,), jnp.int32) for _ in range(4)]
          + [pltpu.VMEM((ECHUNK, 16), jnp.float32) for _ in range(4)]
          + [pltpu.VMEM((ECHUNK, D), jnp.float32) for _ in range(2)]
          + [pltpu.VMEM((ECHUNK, D), jnp.float32) for _ in range(2)]
          + [pltpu.SemaphoreType.DMA for _ in range(8)],
    )
    def _sc_agg(m_hbm, src8_hbm, dst3_hbm, attr3_hbm, zeros_hbm, agg_hbm,
                slab,
                sr0, sr1, sr2, sr3, ds0, ds1, ds2, ds3, at0, at1, at2, at3,
                gb0, gb1, sb0, sb1,
                g0, g1, s0, s1, st0, st1, st2, st3):
        cid = lax.axis_index("c")
        sid = lax.axis_index("s")
        srcs = (sr0, sr1, sr2, sr3)
        dsts = (ds0, ds1, ds2, ds3)
        attrs = (at0, at1, at2, at3)
        gbufs = (gb0, gb1)
        sbufs = (sb0, sb1)
        gsems = (g0, g1)
        ssems = (s0, s1)
        stsems = (st0, st1, st2, st3)

        for j in range(NFB // 2):
            fb = cid * (NFB // 2) + j

            def stage_start(q, tc):
                pltpu.async_copy(src8_hbm.at[fb, sid, tc], srcs[q], stsems[q])
                pltpu.async_copy(dst3_hbm.at[sid, tc], dsts[q], stsems[q])
                pltpu.async_copy(attr3_hbm.at[sid, tc], attrs[q], stsems[q])

            def stage_wait(q, tc):
                pltpu.make_async_copy(src8_hbm.at[fb, sid, tc], srcs[q],
                                      stsems[q]).wait()
                pltpu.make_async_copy(dst3_hbm.at[sid, tc], dsts[q],
                                      stsems[q]).wait()
                pltpu.make_async_copy(attr3_hbm.at[sid, tc], attrs[q],
                                      stsems[q]).wait()


            def gather_start(b, q):
                pltpu.async_copy(m_hbm.at[srcs[q]], gbufs[b], gsems[b])

            def gather_wait(b, q):
                pltpu.make_async_copy(m_hbm.at[srcs[q]], gbufs[b],
                                      gsems[b]).wait()

            def scatter_start(b, q):
                pltpu.async_copy(sbufs[b], slab.at[dsts[q]], ssems[b],
                                 add=True)

            def scatter_wait(b, q):
                pltpu.make_async_copy(sbufs[b], slab.at[dsts[q]],
                                      ssems[b]).wait()

            def mul(b, q):
                gbuf, sbuf, attr = gbufs[b], sbufs[b], attrs[q]

                def edge(e, carry):
                    av = attr[e, :]
                    for c in range(D // 16):
                        sl = pl.ds(c * 16, 16)
                        sbuf[e, sl] = gbuf[e, sl] * av
                    return carry

                lax.fori_loop(0, ECHUNK, edge, 0)

            pltpu.sync_copy(zeros_hbm, slab.at[pl.ds(sid * STRIPE, STRIPE)])
            plsc.subcore_barrier()

            for q in (0, 1):
                stage_start(q, q)
                stage_wait(q, q)
                gather_start(q, q)

            def body(t4, carry):
                for u in range(4):
                    b = u % 2
                    qc = (u + 2) % 4
                    t = 4 * t4 + u
                    gather_wait(b, u)
                    if u < 2:
                        @pl.when(t4 > 0)
                        def _():
                            scatter_wait(b, qc)
                    else:
                        scatter_wait(b, qc)
                    tc = jnp.minimum(t + 2, NCHUNK - 1)
                    stage_start(qc, tc)
                    mul(b, u)
                    scatter_start(b, u)
                    stage_wait(qc, tc)
                    gather_start(b, qc)
                return carry

            lax.fori_loop(0, NCHUNK // 4, body, 0)
            scatter_wait(0, 2)
            scatter_wait(1, 3)
            gather_wait(0, 0)
            gather_wait(1, 1)

            plsc.subcore_barrier()
            pltpu.sync_copy(slab.at[pl.ds(sid * STRIPE, STRIPE)],
                            agg_hbm.at[pl.ds(fb * NP + sid * STRIPE, STRIPE)])
            plsc.subcore_barrier()

    return _sc_agg


def _sc_agg_call(m2, srcp, dstp, attrp, zerosb):
    return _sc_agg_fn()(m2, srcp, dstp, attrp, zerosb)


def _gate_pack(w):
    wt = w.T.reshape(H, 3, H)
    wt = jnp.pad(wt, ((0, HP - H), (0, 0), (0, HP - H)))
    return wt.reshape(HP, 3 * HP)


def _bias_pack(b):
    bp = jnp.pad(b.reshape(3, H), ((0, 0), (0, HP - H))).reshape(3 * HP)
    return jnp.broadcast_to(bp, (8, 3 * HP))


def kernel(x, edge_index, edge_attr, prob, weight, w_ih, w_hh, b_ih, b_hh,
           lin_w, lin_b):
    f32 = jnp.float32
    xp = jnp.pad(x.astype(f32), ((0, NP - N), (0, 0)))
    h = jnp.pad(xp, ((0, 0), (0, HP - D)))
    w0r = jnp.pad(weight[0][:D, :], ((0, 0), (0, HP - H)))
    wnext = {
        1: jnp.pad(weight[1], ((0, HP - H), (0, HP - H))),
        2: jnp.pad(weight[2], ((0, HP - H), (0, HP - H))),
    }
    wih_t = _gate_pack(w_ih)
    whh_t = _gate_pack(w_hh)
    bi = _bias_pack(b_ih)
    bh = _bias_pack(b_hh)
    srcp = jnp.pad(edge_index[0].astype(jnp.int32), (0, EP - E))
    srcp = (srcp[None, :] + (jnp.arange(NFB, dtype=jnp.int32) * NP)[:, None]
            ).reshape(NFB, NTILES, NCHUNK, ECHUNK)
    dstp = jnp.pad(edge_index[1].astype(jnp.int32),
                   (0, EP - E)).reshape(NTILES, NCHUNK, ECHUNK)
    attrp = jnp.pad(edge_attr.astype(f32), (0, EP - E))
    attrp = jnp.broadcast_to(attrp[:, None],
                             (EP, 16)).reshape(NTILES, NCHUNK, ECHUNK, 16)
    zerosb = jnp.zeros((STRIPE, D), f32)
    linp = jnp.zeros((HP, D), f32).at[:H, 0].set(lin_w[0])

    m = _mm0(xp, w0r)
    for layer in (1, 2):
        agg = _sc_agg_call(m.reshape(NFB * NP, D), srcp, dstp, attrp, zerosb)
        h, m = _gru_next(agg.reshape(NFB, NP, D), h, wih_t, whh_t, bi, bh,
                         wnext[layer])
    agg = _sc_agg_call(m.reshape(NFB * NP, D), srcp, dstp, attrp, zerosb)
    out = _gru_final(agg.reshape(NFB, NP, D), h, wih_t, whh_t, bi, bh, linp)
    return out[:N, :1] + lin_b[0]
